# baseline (speedup 1.0000x reference)
import functools

import jax
import jax.numpy as jnp
from jax import lax
from jax.experimental import pallas as pl
from jax.experimental.pallas import tpu as pltpu
from jax.experimental.pallas import tpu_sc as plsc

_INFO = plsc.get_sparse_core_info()
_NC = _INFO.num_cores
_NS = _INFO.num_subcores
_NW = _NC * _NS

_CHUNK = 128
_HB = 16384
_BLK = 2 * _HB


def _transpose_body(in_ref, out_ref):
    x = in_ref[...]
    hidden = in_ref.shape[0]
    a = jnp.concatenate([x[:, :_HB], x[:, _HB:]], axis=0)
    out_ref[...] = a.T


@functools.partial(jax.jit, static_argnames=("hidden", "nblk"))
def _tc_pack(tt, hidden, nblk):
    return pl.pallas_call(
        _transpose_body,
        grid=(nblk,),
        in_specs=[pl.BlockSpec((hidden, _BLK), lambda i: (0, i))],
        out_specs=pl.BlockSpec((_HB, 2 * hidden), lambda i: (i, 0)),
        out_shape=jax.ShapeDtypeStruct((nblk * _HB, 2 * hidden), jnp.float32),
    )(tt)


@functools.partial(jax.jit, static_argnames=("batch", "hidden"))
def _sc_gather(labels3d, tbl, batch, hidden):
    rows_per_w = batch // _NW
    chunks_per_w = rows_per_w // _CHUNK

    mesh = plsc.VectorSubcoreMesh(core_axis_name="c", subcore_axis_name="s")

    @functools.partial(
        pl.kernel,
        mesh=mesh,
        out_type=jax.ShapeDtypeStruct((hidden, batch), jnp.float32),
        scratch_types=[
            pltpu.VMEM((chunks_per_w, _CHUNK), jnp.int32),
            pltpu.VMEM((chunks_per_w, _CHUNK), jnp.int32),
            pltpu.VMEM((rows_per_w, hidden), jnp.float32),
            pltpu.VMEM((hidden, rows_per_w), jnp.float32),
            pltpu.SemaphoreType.DMA,
            pltpu.SemaphoreType.DMA,
        ],
        compiler_params=pltpu.CompilerParams(
            use_tc_tiling_on_sc=False, needs_layout_passes=False
        ),
    )
    def k(lab_hbm, tbl_hbm, out_hbm, lab_v, q_v, rows_v, tbuf_v, sem, sem_o):
        wid = lax.axis_index("s") * _NC + lax.axis_index("c")
        pltpu.sync_copy(lab_hbm.at[wid], lab_v)
        lg = _HB.bit_length() - 1
        for j in range(chunks_per_w):
            for v in range(_CHUNK // 16):
                sl = pl.ds(v * 16, 16)
                r = lab_v[j, sl]
                q_v[j, sl] = (
                    lax.shift_right_logical(r, lg + 1) * (2 * _HB)
                    + 2 * lax.bitwise_and(r, _HB - 1)
                    + lax.bitwise_and(lax.shift_right_logical(r, lg), 1)
                )
        copies = []
        for j in range(chunks_per_w):
            copies.append(
                pltpu.async_copy(
                    tbl_hbm.at[q_v.at[j]],
                    rows_v.at[pl.ds(j * _CHUNK, _CHUNK)],
                    sem,
                )
            )
        for c in copies:
            c.wait()
        lanes = lax.iota(jnp.int32, 16)

        def tgrp(g):
            ridx = lanes + g * 16
            for c in range(hidden):
                tbuf_v[c, pl.ds(g * 16, 16)] = plsc.load_gather(
                    rows_v, [ridx, jnp.full((16,), c, jnp.int32)]
                )

        pl.loop(0, rows_per_w // 16)(tgrp)
        pltpu.sync_copy(
            tbuf_v, out_hbm.at[:, pl.ds(wid * rows_per_w, rows_per_w)]
        )

    return k(labels3d, tbl)


def kernel(labels, table):
    (batch,) = labels.shape
    nrows, hidden = table.shape
    nblk = (nrows + _BLK - 1) // _BLK
    packed = _tc_pack(table.T, hidden, nblk)
    tbl = packed.reshape(-1).reshape(2 * nblk * _HB, hidden)
    lab = labels.astype(jnp.int32).reshape(_NW, batch // _NW // _CHUNK, _CHUNK)
    return _sc_gather(lab, tbl, batch, hidden).T


# speedup vs baseline: 1.0861x; 1.0861x over previous
import functools

import jax
import jax.numpy as jnp
from jax import lax
from jax.experimental import pallas as pl
from jax.experimental.pallas import tpu as pltpu
from jax.experimental.pallas import tpu_sc as plsc

_INFO = plsc.get_sparse_core_info()
_NC = _INFO.num_cores
_NS = _INFO.num_subcores
_NW = _NC * _NS

_CHUNK = 128
_HB = 16384
_BLK = 2 * _HB


def _transpose_body(in_ref, out_ref):
    x = in_ref[...]
    hidden = in_ref.shape[0]
    a = jnp.concatenate([x[:, :_HB], x[:, _HB:]], axis=0)
    eye = jnp.eye(2 * hidden, dtype=jnp.float32)
    dn = (((0,), (0,)), ((), ()))
    out_ref[...] = lax.dot_general(a, eye, dn, preferred_element_type=jnp.float32)


@functools.partial(jax.jit, static_argnames=("hidden", "nblk"))
def _tc_pack(tt, hidden, nblk):
    return pl.pallas_call(
        _transpose_body,
        grid=(nblk,),
        in_specs=[pl.BlockSpec((hidden, _BLK), lambda i: (0, i))],
        out_specs=pl.BlockSpec((_HB, 2 * hidden), lambda i: (i, 0)),
        out_shape=jax.ShapeDtypeStruct((nblk * _HB, 2 * hidden), jnp.float32),
    )(tt)


@functools.partial(jax.jit, static_argnames=("batch", "hidden"))
def _sc_gather(labels3d, tbl, batch, hidden):
    rows_per_w = batch // _NW
    chunks_per_w = rows_per_w // _CHUNK

    mesh = plsc.VectorSubcoreMesh(core_axis_name="c", subcore_axis_name="s")

    @functools.partial(
        pl.kernel,
        mesh=mesh,
        out_type=jax.ShapeDtypeStruct((batch, hidden), jnp.float32),
        scratch_types=[
            pltpu.VMEM((chunks_per_w, _CHUNK), jnp.int32),
            pltpu.VMEM((chunks_per_w, _CHUNK), jnp.int32),
            pltpu.VMEM((rows_per_w, hidden), jnp.float32),
            pltpu.SemaphoreType.DMA,
        ],
        compiler_params=pltpu.CompilerParams(
            use_tc_tiling_on_sc=False, needs_layout_passes=False
        ),
    )
    def k(lab_hbm, tbl_hbm, out_hbm, lab_v, q_v, rows_v, sem):
        wid = lax.axis_index("s") * _NC + lax.axis_index("c")
        pltpu.sync_copy(lab_hbm.at[wid], lab_v)
        lg = _HB.bit_length() - 1
        for j in range(chunks_per_w):
            for v in range(_CHUNK // 16):
                sl = pl.ds(v * 16, 16)
                r = lab_v[j, sl]
                q_v[j, sl] = (
                    lax.shift_right_logical(r, lg + 1) * (2 * _HB)
                    + 2 * lax.bitwise_and(r, _HB - 1)
                    + lax.bitwise_and(lax.shift_right_logical(r, lg), 1)
                )
        copies = []
        for j in range(chunks_per_w):
            copies.append(
                pltpu.async_copy(
                    tbl_hbm.at[q_v.at[j]],
                    rows_v.at[pl.ds(j * _CHUNK, _CHUNK)],
                    sem,
                )
            )
        for c in copies:
            c.wait()
        pltpu.sync_copy(rows_v, out_hbm.at[pl.ds(wid * rows_per_w, rows_per_w)])

    return k(labels3d, tbl)


def kernel(labels, table):
    (batch,) = labels.shape
    nrows, hidden = table.shape
    nblk = (nrows + _BLK - 1) // _BLK
    packed = _tc_pack(table.T, hidden, nblk)
    tbl = packed.reshape(-1).reshape(2 * nblk * _HB, hidden)
    lab = labels.astype(jnp.int32).reshape(_NW, batch // _NW // _CHUNK, _CHUNK)
    return _sc_gather(lab, tbl, batch, hidden)
